# baseline (speedup 1.0000x reference)
import functools

import jax
import jax.numpy as jnp
from jax import lax
from jax.experimental import pallas as pl
from jax.experimental.pallas import tpu as pltpu
from jax.experimental.pallas import tpu_sc as plsc

N = 10000
NP = 10240
EP = 327680
CH = 128
NCH = (EP // 32) // CH
NC, NS, L = 2, 16, 16
RPT = NP // NS


def _mesh():
    return plsc.VectorSubcoreMesh(
        core_axis_name="c", subcore_axis_name="s",
        num_cores=NC, num_subcores=NS)


_SC_PARAMS = pltpu.CompilerParams(needs_layout_passes=False)
_SC_PARAMS_UNTILED = pltpu.CompilerParams(
    needs_layout_passes=False, use_tc_tiling_on_sc=False)


def _rsqrt16(x):
    i = lax.bitcast_convert_type(x, jnp.int32)
    i = jnp.full((L,), 0x5F3759DF, jnp.int32) - lax.shift_right_logical(i, 1)
    y = lax.bitcast_convert_type(i, jnp.float32)
    for _ in range(3):
        y = y * (1.5 - 0.5 * x * y * y)
    return y


def _deg_call(colr, ewr):
    @functools.partial(
        pl.kernel,
        out_type=jax.ShapeDtypeStruct((NC, NP), jnp.float32),
        mesh=_mesh(),
        compiler_params=_SC_PARAMS,
        scratch_types=[
            pltpu.VMEM((NCH, CH), jnp.int32),
            pltpu.VMEM((NCH, CH), jnp.float32),
            pltpu.VMEM((RPT,), jnp.float32),
            pltpu.VMEM_SHARED((NP,), jnp.float32),
        ],
    )
    def deg_kernel(col_hbm, ew_hbm, out_hbm, col_v, ew_v, zb_v, acc_sh):
        cid = lax.axis_index("c")
        sid = lax.axis_index("s")
        wid = cid * NS + sid
        pltpu.sync_copy(col_hbm.at[pl.ds(wid * NCH, NCH)], col_v)
        pltpu.sync_copy(ew_hbm.at[pl.ds(wid * NCH, NCH)], ew_v)

        def zb(k, carry):
            zb_v[pl.ds(k * L, L)] = jnp.zeros((L,), jnp.float32)
            return carry
        lax.fori_loop(0, RPT // L, zb, 0)
        pltpu.sync_copy(zb_v, acc_sh.at[pl.ds(sid * RPT, RPT)])
        plsc.subcore_barrier()

        def chunk(ch, carry):
            pltpu.sync_copy(ew_v.at[ch], acc_sh.at[col_v.at[ch]], add=True)
            return carry
        lax.fori_loop(0, NCH, chunk, 0)
        plsc.subcore_barrier()
        pltpu.sync_copy(acc_sh.at[pl.ds(sid * RPT, RPT)],
                        out_hbm.at[cid, pl.ds(sid * RPT, RPT)])

    return deg_kernel(colr, ewr)


def _norm_call(rowr, colr, ewr, deg):
    @functools.partial(
        pl.kernel,
        out_type=jax.ShapeDtypeStruct((EP // CH, CH), jnp.float32),
        mesh=_mesh(),
        compiler_params=_SC_PARAMS,
        scratch_types=[
            pltpu.VMEM((NCH, CH), jnp.int32),
            pltpu.VMEM((NCH, CH), jnp.int32),
            pltpu.VMEM((NCH, CH), jnp.float32),
            pltpu.VMEM((NC, NP), jnp.float32),
            pltpu.VMEM((NP,), jnp.float32),
        ],
    )
    def norm_kernel(row_hbm, col_hbm, ew_hbm, deg_hbm, s_hbm,
                    row_v, col_v, ew_v, deg_v, dinv_v):
        cid = lax.axis_index("c")
        sid = lax.axis_index("s")
        wid = cid * NS + sid
        pltpu.sync_copy(row_hbm.at[pl.ds(wid * NCH, NCH)], row_v)
        pltpu.sync_copy(col_hbm.at[pl.ds(wid * NCH, NCH)], col_v)
        pltpu.sync_copy(ew_hbm.at[pl.ds(wid * NCH, NCH)], ew_v)
        pltpu.sync_copy(deg_hbm, deg_v)

        def dbody(k, carry):
            sl = pl.ds(k * L, L)
            d = deg_v[0, sl] + deg_v[1, sl] + 1.0
            dinv_v[sl] = _rsqrt16(d)
            return carry
        lax.fori_loop(0, NP // L, dbody, 0)

        def chunk(ch, carry):
            for sub in range(CH // L):
                sl = pl.ds(sub * L, L)
                rr = row_v[ch, sl]
                cc = col_v[ch, sl]
                w = ew_v[ch, sl]
                ew_v[ch, sl] = (plsc.load_gather(dinv_v, [rr]) * w *
                                plsc.load_gather(dinv_v, [cc]))
            return carry
        lax.fori_loop(0, NCH, chunk, 0)
        pltpu.sync_copy(ew_v, s_hbm.at[pl.ds(wid * NCH, NCH)])

    return norm_kernel(rowr, colr, ewr, deg)


def _agg_call(F, g, rowr, colr, sr):
    @functools.partial(
        pl.kernel,
        out_type=jax.ShapeDtypeStruct((NC, NP, F), jnp.float32),
        mesh=_mesh(),
        compiler_params=_SC_PARAMS if F % 128 == 0 else _SC_PARAMS_UNTILED,
        scratch_types=[
            pltpu.VMEM((NCH, CH), jnp.int32),
            pltpu.VMEM((NCH, CH), jnp.int32),
            pltpu.VMEM((NCH, CH), jnp.float32),
            pltpu.VMEM((2, CH, F), jnp.float32),
            pltpu.VMEM((2, CH, F), jnp.float32),
            pltpu.VMEM_SHARED((NP, F), jnp.float32),
            pltpu.SemaphoreType.DMA,
            pltpu.SemaphoreType.DMA,
            pltpu.SemaphoreType.DMA,
            pltpu.SemaphoreType.DMA,
        ],
    )
    def agg_kernel(g_hbm, row_hbm, col_hbm, s_hbm, out_hbm,
                   row_v, col_v, s_v, gbuf, sbuf, acc_sh,
                   sg0, sg1, ss0, ss1):
        cid = lax.axis_index("c")
        sid = lax.axis_index("s")
        wid = cid * NS + sid
        semg = (sg0, sg1)
        sems = (ss0, ss1)

        pltpu.sync_copy(row_hbm.at[pl.ds(wid * NCH, NCH)], row_v)
        pltpu.sync_copy(col_hbm.at[pl.ds(wid * NCH, NCH)], col_v)
        pltpu.sync_copy(s_hbm.at[pl.ds(wid * NCH, NCH)], s_v)

        def zrow(r, carry):
            for gg in range(F // L):
                gbuf[0, r, pl.ds(gg * L, L)] = jnp.zeros((L,), jnp.float32)
            return carry
        lax.fori_loop(0, CH, zrow, 0)
        for k in range(RPT // CH):
            pltpu.sync_copy(gbuf.at[0],
                            acc_sh.at[pl.ds(sid * RPT + k * CH, CH)])
        plsc.subcore_barrier()

        def issue_gather(b, ch):
            pltpu.async_copy(g_hbm.at[row_v.at[ch]], gbuf.at[b], semg[b])

        def wait_gather(b, ch):
            pltpu.make_async_copy(
                g_hbm.at[row_v.at[ch]], gbuf.at[b], semg[b]).wait()

        def issue_scatter(b, ch):
            pltpu.async_copy(sbuf.at[b], acc_sh.at[col_v.at[ch]],
                             sems[b], add=True)

        def wait_scatter(b, ch):
            pltpu.make_async_copy(
                sbuf.at[b], acc_sh.at[col_v.at[ch]], sems[b]).wait()

        def scale(b, ch):
            def rbody(r, carry):
                for rr in range(4):
                    row = r * 4 + rr
                    sb = plsc.load_gather(
                        s_v, [jnp.full((L,), ch, jnp.int32),
                              jnp.full((L,), row, jnp.int32)])
                    for gg in range(F // L):
                        sl = pl.ds(gg * L, L)
                        sbuf[b, row, sl] = gbuf[b, row, sl] * sb
                return carry
            lax.fori_loop(0, CH // 4, rbody, 0)

        for b in range(2):
            issue_gather(b, b)
        for b in range(2):
            wait_gather(b, b)
            scale(b, b)
            issue_scatter(b, b)
            issue_gather(b, b + 2)

        def step(k, carry):
            for b in range(2):
                ch = 2 * k + b
                wait_gather(b, ch)
                wait_scatter(b, ch - 2)
                scale(b, ch)
                issue_scatter(b, ch)
                issue_gather(b, ch + 2)
            return carry
        lax.fori_loop(1, NCH // 2 - 1, step, 0)

        for b in range(2):
            ch = NCH - 2 + b
            wait_gather(b, ch)
            wait_scatter(b, ch - 2)
            scale(b, ch)
            issue_scatter(b, ch)
        for b in range(2):
            wait_scatter(b, NCH - 2 + b)

        plsc.subcore_barrier()
        pltpu.sync_copy(acc_sh.at[pl.ds(sid * RPT, RPT)],
                        out_hbm.at[cid, pl.ds(sid * RPT, RPT)])

    return agg_kernel(g, rowr, colr, sr)


def _mm_call(x, w):
    def body(x_ref, w_ref, o_ref):
        o_ref[...] = jnp.dot(x_ref[...], w_ref[...],
                             preferred_element_type=jnp.float32)
    return pl.pallas_call(
        body,
        out_shape=jax.ShapeDtypeStruct((x.shape[0], w.shape[1]), jnp.float32),
    )(x, w)


def _mid_call(degT, aa0, aa1, ab0, ab1, g1, b1r, W2p):
    def body(d_ref, aa0_ref, aa1_ref, ab0_ref, ab1_ref, g_ref, b_ref,
             w_ref, o_ref):
        inv = 1.0 / (d_ref[:, 0:1] + d_ref[:, 1:2] + 1.0)
        agg = jnp.concatenate(
            [aa0_ref[...] + aa1_ref[...], ab0_ref[...] + ab1_ref[...]],
            axis=1)
        z = agg + g_ref[...] * inv + b_ref[...]
        h = jnp.maximum(z, 0.0)
        o_ref[...] = jnp.dot(h, w_ref[...],
                             preferred_element_type=jnp.float32)
    return pl.pallas_call(
        body,
        out_shape=jax.ShapeDtypeStruct((NP, W2p.shape[1]), jnp.float32),
    )(degT, aa0, aa1, ab0, ab1, g1, b1r, W2p)


def _final_call(degT, a0, a1, g2, b2r):
    F2 = b2r.shape[1]
    def body(d_ref, a0_ref, a1_ref, g_ref, b_ref, o_ref):
        inv = 1.0 / (d_ref[:, 0:1] + d_ref[:, 1:2] + 1.0)
        z = (a0_ref[...] + a1_ref[...] + g_ref[...] * inv)[:, :F2] + b_ref[...]
        m = jnp.max(z, axis=1, keepdims=True)
        e = jnp.exp(z - m)
        s = jnp.sum(e, axis=1, keepdims=True)
        o_ref[...] = z - m - jnp.log(s)
    return pl.pallas_call(
        body,
        out_shape=jax.ShapeDtypeStruct((NP, F2), jnp.float32),
    )(degT, a0, a1, g2, b2r)


def kernel(x, edge_index, edge_weight, W1, b1, W2, b2):
    row = edge_index[0].astype(jnp.int32)
    col = edge_index[1].astype(jnp.int32)
    ew = edge_weight.astype(jnp.float32)
    pad = EP - row.shape[0]
    pidx = (jnp.arange(pad, dtype=jnp.int32) * 37) % N
    rowp = jnp.concatenate([row, pidx]).reshape(EP // CH, CH)
    colp = jnp.concatenate([col, pidx]).reshape(EP // CH, CH)
    ewp = jnp.concatenate([ew, jnp.zeros((pad,), jnp.float32)]
                          ).reshape(EP // CH, CH)
    xp = jnp.concatenate(
        [x, jnp.zeros((NP - N, x.shape[1]), jnp.float32)], axis=0)
    F2P = 64
    W2p = jnp.concatenate(
        [W2, jnp.zeros((W2.shape[0], F2P - W2.shape[1]), jnp.float32)], axis=1)

    deg = _deg_call(colp, ewp)
    degT = deg.T
    sp = _norm_call(rowp, colp, ewp, deg)
    g1 = _mm_call(xp, W1)
    a1a = _agg_call(64, g1[:, :64], rowp, colp, sp)
    a1b = _agg_call(64, g1[:, 64:], rowp, colp, sp)
    g2 = _mid_call(degT, a1a[0], a1a[1], a1b[0], a1b[1], g1,
                   b1.reshape(1, -1), W2p)
    agg2 = _agg_call(F2P, g2, rowp, colp, sp)
    out = _final_call(degT, agg2[0], agg2[1], g2, b2.reshape(1, -1))
    return out[:N]


# speedup vs baseline: 1.9383x; 1.9383x over previous
import functools

import jax
import jax.numpy as jnp
from jax import lax
from jax.experimental import pallas as pl
from jax.experimental.pallas import tpu as pltpu
from jax.experimental.pallas import tpu_sc as plsc

N = 10000
NP = 10240
EP = 327680
CH = 128
NCH = (EP // 32) // CH
NC, NS, L = 2, 16, 16
RPT = NP // NS


def _mesh():
    return plsc.VectorSubcoreMesh(
        core_axis_name="c", subcore_axis_name="s",
        num_cores=NC, num_subcores=NS)


_SC_PARAMS = pltpu.CompilerParams(needs_layout_passes=False)
_SC_PARAMS_UNTILED = pltpu.CompilerParams(
    needs_layout_passes=False, use_tc_tiling_on_sc=False)


def _rsqrt16(x):
    i = lax.bitcast_convert_type(x, jnp.int32)
    i = jnp.full((L,), 0x5F3759DF, jnp.int32) - lax.shift_right_logical(i, 1)
    y = lax.bitcast_convert_type(i, jnp.float32)
    for _ in range(3):
        y = y * (1.5 - 0.5 * x * y * y)
    return y


def _deg_call(colr, ewr):
    @functools.partial(
        pl.kernel,
        out_type=jax.ShapeDtypeStruct((NC, NP), jnp.float32),
        mesh=_mesh(),
        compiler_params=_SC_PARAMS,
        scratch_types=[
            pltpu.VMEM((NCH, CH), jnp.int32),
            pltpu.VMEM((NCH, CH), jnp.float32),
            pltpu.VMEM((RPT,), jnp.float32),
            pltpu.VMEM_SHARED((NP,), jnp.float32),
        ],
    )
    def deg_kernel(col_hbm, ew_hbm, out_hbm, col_v, ew_v, zb_v, acc_sh):
        cid = lax.axis_index("c")
        sid = lax.axis_index("s")
        wid = cid * NS + sid
        pltpu.sync_copy(col_hbm.at[pl.ds(wid * NCH, NCH)], col_v)
        pltpu.sync_copy(ew_hbm.at[pl.ds(wid * NCH, NCH)], ew_v)

        def zb(k, carry):
            zb_v[pl.ds(k * L, L)] = jnp.zeros((L,), jnp.float32)
            return carry
        lax.fori_loop(0, RPT // L, zb, 0)
        pltpu.sync_copy(zb_v, acc_sh.at[pl.ds(sid * RPT, RPT)])
        plsc.subcore_barrier()

        def chunk(ch, carry):
            pltpu.sync_copy(ew_v.at[ch], acc_sh.at[col_v.at[ch]], add=True)
            return carry
        lax.fori_loop(0, NCH, chunk, 0)
        plsc.subcore_barrier()
        pltpu.sync_copy(acc_sh.at[pl.ds(sid * RPT, RPT)],
                        out_hbm.at[cid, pl.ds(sid * RPT, RPT)])

    return deg_kernel(colr, ewr)


def _norm_call(rowr, colr, ewr, deg):
    @functools.partial(
        pl.kernel,
        out_type=jax.ShapeDtypeStruct((EP // CH, CH), jnp.float32),
        mesh=_mesh(),
        compiler_params=_SC_PARAMS,
        scratch_types=[
            pltpu.VMEM((NCH, CH), jnp.int32),
            pltpu.VMEM((NCH, CH), jnp.int32),
            pltpu.VMEM((NCH, CH), jnp.float32),
            pltpu.VMEM((NC, NP), jnp.float32),
            pltpu.VMEM((NP,), jnp.float32),
        ],
    )
    def norm_kernel(row_hbm, col_hbm, ew_hbm, deg_hbm, s_hbm,
                    row_v, col_v, ew_v, deg_v, dinv_v):
        cid = lax.axis_index("c")
        sid = lax.axis_index("s")
        wid = cid * NS + sid
        pltpu.sync_copy(row_hbm.at[pl.ds(wid * NCH, NCH)], row_v)
        pltpu.sync_copy(col_hbm.at[pl.ds(wid * NCH, NCH)], col_v)
        pltpu.sync_copy(ew_hbm.at[pl.ds(wid * NCH, NCH)], ew_v)
        pltpu.sync_copy(deg_hbm, deg_v)

        def dbody(k, carry):
            sl = pl.ds(k * L, L)
            d = deg_v[0, sl] + deg_v[1, sl] + 1.0
            dinv_v[sl] = _rsqrt16(d)
            return carry
        lax.fori_loop(0, NP // L, dbody, 0)

        def chunk(ch, carry):
            for sub in range(CH // L):
                sl = pl.ds(sub * L, L)
                rr = row_v[ch, sl]
                cc = col_v[ch, sl]
                w = ew_v[ch, sl]
                ew_v[ch, sl] = (plsc.load_gather(dinv_v, [rr]) * w *
                                plsc.load_gather(dinv_v, [cc]))
            return carry
        lax.fori_loop(0, NCH, chunk, 0)
        pltpu.sync_copy(ew_v, s_hbm.at[pl.ds(wid * NCH, NCH)])

    return norm_kernel(rowr, colr, ewr, deg)


def _agg_call(F, g, rowr, colr, sr):
    @functools.partial(
        pl.kernel,
        out_type=jax.ShapeDtypeStruct((NC, NP, F), jnp.float32),
        mesh=_mesh(),
        compiler_params=_SC_PARAMS if F % 128 == 0 else _SC_PARAMS_UNTILED,
        scratch_types=[
            pltpu.VMEM((NCH, CH), jnp.int32),
            pltpu.VMEM((NCH, CH), jnp.int32),
            pltpu.VMEM((NCH, CH), jnp.float32),
            pltpu.VMEM((2, CH, F), jnp.float32),
            pltpu.VMEM((2, CH, F), jnp.float32),
            pltpu.VMEM_SHARED((NP, F), jnp.float32),
            pltpu.SemaphoreType.DMA,
            pltpu.SemaphoreType.DMA,
            pltpu.SemaphoreType.DMA,
            pltpu.SemaphoreType.DMA,
        ],
    )
    def agg_kernel(g_hbm, row_hbm, col_hbm, s_hbm, out_hbm,
                   row_v, col_v, s_v, gbuf, sbuf, acc_sh,
                   sg0, sg1, ss0, ss1):
        cid = lax.axis_index("c")
        sid = lax.axis_index("s")
        wid = cid * NS + sid
        semg = (sg0, sg1)
        sems = (ss0, ss1)

        pltpu.sync_copy(row_hbm.at[pl.ds(wid * NCH, NCH)], row_v)
        pltpu.sync_copy(col_hbm.at[pl.ds(wid * NCH, NCH)], col_v)
        pltpu.sync_copy(s_hbm.at[pl.ds(wid * NCH, NCH)], s_v)

        def zrow(r, carry):
            for gg in range(F // L):
                gbuf[0, r, pl.ds(gg * L, L)] = jnp.zeros((L,), jnp.float32)
            return carry
        lax.fori_loop(0, CH, zrow, 0)
        for k in range(RPT // CH):
            pltpu.sync_copy(gbuf.at[0],
                            acc_sh.at[pl.ds(sid * RPT + k * CH, CH)])
        plsc.subcore_barrier()

        def issue_gather(b, ch):
            pltpu.async_copy(g_hbm.at[row_v.at[ch]], gbuf.at[b], semg[b])

        def wait_gather(b, ch):
            pltpu.make_async_copy(
                g_hbm.at[row_v.at[ch]], gbuf.at[b], semg[b]).wait()

        def issue_scatter(b, ch):
            pltpu.async_copy(sbuf.at[b], acc_sh.at[col_v.at[ch]],
                             sems[b], add=True)

        def wait_scatter(b, ch):
            pltpu.make_async_copy(
                sbuf.at[b], acc_sh.at[col_v.at[ch]], sems[b]).wait()

        def scale(b, ch):
            R = 4

            def rbody(r, carry):
                rows = [r * R + rr for rr in range(R)]
                sbs = [plsc.load_gather(
                    s_v, [jnp.full((L,), ch, jnp.int32),
                          jnp.full((L,), row, jnp.int32)])
                       for row in rows]
                vals = [[gbuf[b, row, pl.ds(gg * L, L)]
                         for gg in range(F // L)] for row in rows]
                for rr, row in enumerate(rows):
                    for gg in range(F // L):
                        sbuf[b, row, pl.ds(gg * L, L)] = vals[rr][gg] * sbs[rr]
                return carry
            lax.fori_loop(0, CH // R, rbody, 0)

        for b in range(2):
            issue_gather(b, b)
        for b in range(2):
            wait_gather(b, b)
            scale(b, b)
            issue_scatter(b, b)
            issue_gather(b, b + 2)

        def step(k, carry):
            for b in range(2):
                ch = 2 * k + b
                wait_gather(b, ch)
                wait_scatter(b, ch - 2)
                scale(b, ch)
                issue_scatter(b, ch)
                issue_gather(b, ch + 2)
            return carry
        lax.fori_loop(1, NCH // 2 - 1, step, 0)

        for b in range(2):
            ch = NCH - 2 + b
            wait_gather(b, ch)
            wait_scatter(b, ch - 2)
            scale(b, ch)
            issue_scatter(b, ch)
        for b in range(2):
            wait_scatter(b, NCH - 2 + b)

        plsc.subcore_barrier()
        pltpu.sync_copy(acc_sh.at[pl.ds(sid * RPT, RPT)],
                        out_hbm.at[cid, pl.ds(sid * RPT, RPT)])

    return agg_kernel(g, rowr, colr, sr)


def _mm_call(x, w):
    def body(x_ref, w_ref, o_ref):
        o_ref[...] = jnp.dot(x_ref[...], w_ref[...],
                             preferred_element_type=jnp.float32)
    return pl.pallas_call(
        body,
        out_shape=jax.ShapeDtypeStruct((x.shape[0], w.shape[1]), jnp.float32),
    )(x, w)


def _mid_call(degT, aa0, aa1, ab0, ab1, g1, b1r, W2p):
    def body(d_ref, aa0_ref, aa1_ref, ab0_ref, ab1_ref, g_ref, b_ref,
             w_ref, o_ref):
        inv = 1.0 / (d_ref[:, 0:1] + d_ref[:, 1:2] + 1.0)
        agg = jnp.concatenate(
            [aa0_ref[...] + aa1_ref[...], ab0_ref[...] + ab1_ref[...]],
            axis=1)
        z = agg + g_ref[...] * inv + b_ref[...]
        h = jnp.maximum(z, 0.0)
        o_ref[...] = jnp.dot(h, w_ref[...],
                             preferred_element_type=jnp.float32)
    return pl.pallas_call(
        body,
        out_shape=jax.ShapeDtypeStruct((NP, W2p.shape[1]), jnp.float32),
    )(degT, aa0, aa1, ab0, ab1, g1, b1r, W2p)


def _final_call(degT, a0, a1, g2, b2r):
    F2 = b2r.shape[1]
    def body(d_ref, a0_ref, a1_ref, g_ref, b_ref, o_ref):
        inv = 1.0 / (d_ref[:, 0:1] + d_ref[:, 1:2] + 1.0)
        z = (a0_ref[...] + a1_ref[...] + g_ref[...] * inv)[:, :F2] + b_ref[...]
        m = jnp.max(z, axis=1, keepdims=True)
        e = jnp.exp(z - m)
        s = jnp.sum(e, axis=1, keepdims=True)
        o_ref[...] = z - m - jnp.log(s)
    return pl.pallas_call(
        body,
        out_shape=jax.ShapeDtypeStruct((NP, F2), jnp.float32),
    )(degT, a0, a1, g2, b2r)


def kernel(x, edge_index, edge_weight, W1, b1, W2, b2):
    row = edge_index[0].astype(jnp.int32)
    col = edge_index[1].astype(jnp.int32)
    ew = edge_weight.astype(jnp.float32)
    pad = EP - row.shape[0]
    pidx = (jnp.arange(pad, dtype=jnp.int32) * 37) % N
    rowp = jnp.concatenate([row, pidx]).reshape(EP // CH, CH)
    colp = jnp.concatenate([col, pidx]).reshape(EP // CH, CH)
    ewp = jnp.concatenate([ew, jnp.zeros((pad,), jnp.float32)]
                          ).reshape(EP // CH, CH)
    xp = jnp.concatenate(
        [x, jnp.zeros((NP - N, x.shape[1]), jnp.float32)], axis=0)
    F2P = 64
    W2p = jnp.concatenate(
        [W2, jnp.zeros((W2.shape[0], F2P - W2.shape[1]), jnp.float32)], axis=1)

    deg = _deg_call(colp, ewp)
    degT = deg.T
    sp = _norm_call(rowp, colp, ewp, deg)
    g1 = _mm_call(xp, W1)
    a1a = _agg_call(64, g1[:, :64], rowp, colp, sp)
    a1b = _agg_call(64, g1[:, 64:], rowp, colp, sp)
    g2 = _mid_call(degT, a1a[0], a1a[1], a1b[0], a1b[1], g1,
                   b1.reshape(1, -1), W2p)
    agg2 = _agg_call(F2P, g2, rowp, colp, sp)
    out = _final_call(degT, agg2[0], agg2[1], g2, b2.reshape(1, -1))
    return out[:N]


# speedup vs baseline: 2.0727x; 1.0694x over previous
import functools

import jax
import jax.numpy as jnp
from jax import lax
from jax.experimental import pallas as pl
from jax.experimental.pallas import tpu as pltpu
from jax.experimental.pallas import tpu_sc as plsc

N = 10000
NP = 10240
EP = 327680
CH = 128
NCH = (EP // 32) // CH
NC, NS, L = 2, 16, 16
RPT = NP // NS


def _mesh():
    return plsc.VectorSubcoreMesh(
        core_axis_name="c", subcore_axis_name="s",
        num_cores=NC, num_subcores=NS)


_SC_PARAMS = pltpu.CompilerParams(needs_layout_passes=False)
_SC_PARAMS_UNTILED = pltpu.CompilerParams(
    needs_layout_passes=False, use_tc_tiling_on_sc=False)


def _rsqrt16(x):
    i = lax.bitcast_convert_type(x, jnp.int32)
    i = jnp.full((L,), 0x5F3759DF, jnp.int32) - lax.shift_right_logical(i, 1)
    y = lax.bitcast_convert_type(i, jnp.float32)
    for _ in range(3):
        y = y * (1.5 - 0.5 * x * y * y)
    return y


def _prep_call(rowr, colr, ewr):
    DPW = (EP // CH) // NS

    @functools.partial(
        pl.kernel,
        out_type=(jax.ShapeDtypeStruct((EP // CH, CH), jnp.float32),
                  jax.ShapeDtypeStruct((NP,), jnp.float32)),
        mesh=_mesh(),
        compiler_params=_SC_PARAMS,
        scratch_types=[
            pltpu.VMEM((DPW, CH), jnp.int32),
            pltpu.VMEM((DPW, CH), jnp.float32),
            pltpu.VMEM((NCH, CH), jnp.int32),
            pltpu.VMEM((NCH, CH), jnp.int32),
            pltpu.VMEM((NCH, CH), jnp.float32),
            pltpu.VMEM((NP,), jnp.float32),
            pltpu.VMEM((NP,), jnp.float32),
            pltpu.VMEM((RPT,), jnp.float32),
            pltpu.VMEM_SHARED((NP,), jnp.float32),
            pltpu.SemaphoreType.DMA,
        ],
    )
    def prep_kernel(row_hbm, col_hbm, ew_hbm, s_hbm, deg_hbm,
                    dcol_v, dew_v, row_v, col_v, ew_v, deg_v, dinv_v, zb_v,
                    acc_sh, sem):
        cid = lax.axis_index("c")
        sid = lax.axis_index("s")
        wid = cid * NS + sid

        pltpu.sync_copy(col_hbm.at[pl.ds(sid * DPW, DPW)], dcol_v)
        pltpu.sync_copy(ew_hbm.at[pl.ds(sid * DPW, DPW)], dew_v)

        def zb(k, carry):
            zb_v[pl.ds(k * L, L)] = jnp.zeros((L,), jnp.float32)
            return carry
        lax.fori_loop(0, RPT // L, zb, 0)
        pltpu.sync_copy(zb_v, acc_sh.at[pl.ds(sid * RPT, RPT)])
        plsc.subcore_barrier()

        K = 8

        def dchunk(k, carry):
            for j in range(K):
                pltpu.async_copy(dew_v.at[k * K + j],
                                 acc_sh.at[dcol_v.at[k * K + j]], sem,
                                 add=True)
            for j in range(K):
                pltpu.make_async_copy(
                    dew_v.at[k * K + j],
                    acc_sh.at[dcol_v.at[k * K + j]], sem).wait()
            return carry
        lax.fori_loop(0, DPW // K, dchunk, 0)
        plsc.subcore_barrier()

        @pl.when(cid == 0)
        def _():
            pltpu.sync_copy(acc_sh.at[pl.ds(sid * RPT, RPT)],
                            deg_hbm.at[pl.ds(sid * RPT, RPT)])

        pltpu.sync_copy(acc_sh, deg_v)
        pltpu.sync_copy(row_hbm.at[pl.ds(wid * NCH, NCH)], row_v)
        pltpu.sync_copy(col_hbm.at[pl.ds(wid * NCH, NCH)], col_v)
        pltpu.sync_copy(ew_hbm.at[pl.ds(wid * NCH, NCH)], ew_v)

        def dbody(k, carry):
            sl = pl.ds(k * L, L)
            d = deg_v[sl] + 1.0
            dinv_v[sl] = _rsqrt16(d)
            return carry
        lax.fori_loop(0, NP // L, dbody, 0)

        def chunk(ch, carry):
            for sub in range(CH // L):
                sl = pl.ds(sub * L, L)
                rr = row_v[ch, sl]
                cc = col_v[ch, sl]
                w = ew_v[ch, sl]
                ew_v[ch, sl] = (plsc.load_gather(dinv_v, [rr]) * w *
                                plsc.load_gather(dinv_v, [cc]))
            return carry
        lax.fori_loop(0, NCH, chunk, 0)
        pltpu.sync_copy(ew_v, s_hbm.at[pl.ds(wid * NCH, NCH)])

    return prep_kernel(rowr, colr, ewr)


def _agg_call(tables, rowr, colr, sr):
    NT = len(tables)
    F = 64

    @functools.partial(
        pl.kernel,
        out_type=jax.ShapeDtypeStruct((NT, NC, NP, F), jnp.float32),
        mesh=_mesh(),
        compiler_params=_SC_PARAMS_UNTILED,
        scratch_types=[
            pltpu.VMEM((NCH, CH), jnp.int32),
            pltpu.VMEM((NCH, CH), jnp.int32),
            pltpu.VMEM((NCH, CH), jnp.float32),
            pltpu.VMEM((2, CH, F), jnp.float32),
            pltpu.VMEM((2, CH, F), jnp.float32),
            pltpu.VMEM_SHARED((NP, F), jnp.float32),
            pltpu.SemaphoreType.DMA,
            pltpu.SemaphoreType.DMA,
            pltpu.SemaphoreType.DMA,
            pltpu.SemaphoreType.DMA,
        ],
    )
    def agg_kernel(*refs):
        g_hbms = refs[:NT]
        row_hbm, col_hbm, s_hbm, out_hbm = refs[NT:NT + 4]
        (row_v, col_v, s_v, gbuf, sbuf, acc_sh,
         sg0, sg1, ss0, ss1) = refs[NT + 4:]
        cid = lax.axis_index("c")
        sid = lax.axis_index("s")
        wid = cid * NS + sid
        semg = (sg0, sg1)
        sems = (ss0, ss1)

        pltpu.sync_copy(row_hbm.at[pl.ds(wid * NCH, NCH)], row_v)
        pltpu.sync_copy(col_hbm.at[pl.ds(wid * NCH, NCH)], col_v)
        pltpu.sync_copy(s_hbm.at[pl.ds(wid * NCH, NCH)], s_v)

        def issue_gather(g_hbm, b, ch):
            pltpu.async_copy(g_hbm.at[row_v.at[ch]], gbuf.at[b], semg[b])

        def wait_gather(g_hbm, b, ch):
            pltpu.make_async_copy(
                g_hbm.at[row_v.at[ch]], gbuf.at[b], semg[b]).wait()

        def issue_scatter(b, ch):
            pltpu.async_copy(sbuf.at[b], acc_sh.at[col_v.at[ch]],
                             sems[b], add=True)

        def wait_scatter(b, ch):
            pltpu.make_async_copy(
                sbuf.at[b], acc_sh.at[col_v.at[ch]], sems[b]).wait()

        def scale(b, ch):
            R = 4

            def rbody(r, carry):
                rows = [r * R + rr for rr in range(R)]
                sbs = [plsc.load_gather(
                    s_v, [jnp.full((L,), ch, jnp.int32),
                          jnp.full((L,), row, jnp.int32)])
                       for row in rows]
                vals = [[gbuf[b, row, pl.ds(gg * L, L)]
                         for gg in range(F // L)] for row in rows]
                for rr, row in enumerate(rows):
                    for gg in range(F // L):
                        sbuf[b, row, pl.ds(gg * L, L)] = vals[rr][gg] * sbs[rr]
                return carry
            lax.fori_loop(0, CH // R, rbody, 0)

        for t, g_hbm in enumerate(g_hbms):
            def zrow(r, carry):
                for gg in range(F // L):
                    gbuf[0, r, pl.ds(gg * L, L)] = jnp.zeros((L,),
                                                            jnp.float32)
                return carry
            lax.fori_loop(0, CH, zrow, 0)
            for k in range(RPT // CH):
                pltpu.sync_copy(gbuf.at[0],
                                acc_sh.at[pl.ds(sid * RPT + k * CH, CH)])
            plsc.subcore_barrier()

            for b in range(2):
                issue_gather(g_hbm, b, b)
            for b in range(2):
                wait_gather(g_hbm, b, b)
                scale(b, b)
                issue_scatter(b, b)
                issue_gather(g_hbm, b, b + 2)

            def step(k, carry):
                for b in range(2):
                    ch = 2 * k + b
                    wait_gather(g_hbm, b, ch)
                    wait_scatter(b, ch - 2)
                    scale(b, ch)
                    issue_scatter(b, ch)
                    issue_gather(g_hbm, b, ch + 2)
                return carry
            lax.fori_loop(1, NCH // 2 - 1, step, 0)

            for b in range(2):
                ch = NCH - 2 + b
                wait_gather(g_hbm, b, ch)
                wait_scatter(b, ch - 2)
                scale(b, ch)
                issue_scatter(b, ch)
            for b in range(2):
                wait_scatter(b, NCH - 2 + b)

            plsc.subcore_barrier()
            pltpu.sync_copy(acc_sh.at[pl.ds(sid * RPT, RPT)],
                            out_hbm.at[t, cid, pl.ds(sid * RPT, RPT)])
            if t + 1 < NT:
                plsc.subcore_barrier()

    return agg_kernel(*tables, rowr, colr, sr)


def _mm_call(x, w):
    def body(x_ref, w_ref, o_ref):
        o_ref[...] = jnp.dot(x_ref[...], w_ref[...],
                             preferred_element_type=jnp.float32)
    return pl.pallas_call(
        body,
        out_shape=jax.ShapeDtypeStruct((x.shape[0], w.shape[1]), jnp.float32),
    )(x, w)


def _mid_call(degc, aa0, aa1, ab0, ab1, g1, b1r, W2p):
    def body(d_ref, aa0_ref, aa1_ref, ab0_ref, ab1_ref, g_ref, b_ref,
             w_ref, o_ref):
        inv = 1.0 / (d_ref[...] + 1.0)
        agg = jnp.concatenate(
            [aa0_ref[...] + aa1_ref[...], ab0_ref[...] + ab1_ref[...]],
            axis=1)
        z = agg + g_ref[...] * inv + b_ref[...]
        h = jnp.maximum(z, 0.0)
        o_ref[...] = jnp.dot(h, w_ref[...],
                             preferred_element_type=jnp.float32)
    return pl.pallas_call(
        body,
        out_shape=jax.ShapeDtypeStruct((NP, W2p.shape[1]), jnp.float32),
    )(degc, aa0, aa1, ab0, ab1, g1, b1r, W2p)


def _final_call(degc, a0, a1, g2, b2r):
    F2 = b2r.shape[1]
    def body(d_ref, a0_ref, a1_ref, g_ref, b_ref, o_ref):
        inv = 1.0 / (d_ref[...] + 1.0)
        z = (a0_ref[...] + a1_ref[...] + g_ref[...] * inv)[:, :F2] + b_ref[...]
        m = jnp.max(z, axis=1, keepdims=True)
        e = jnp.exp(z - m)
        s = jnp.sum(e, axis=1, keepdims=True)
        o_ref[...] = z - m - jnp.log(s)
    return pl.pallas_call(
        body,
        out_shape=jax.ShapeDtypeStruct((NP, F2), jnp.float32),
    )(degc, a0, a1, g2, b2r)


def kernel(x, edge_index, edge_weight, W1, b1, W2, b2):
    row = edge_index[0].astype(jnp.int32)
    col = edge_index[1].astype(jnp.int32)
    ew = edge_weight.astype(jnp.float32)
    pad = EP - row.shape[0]
    pidx = (jnp.arange(pad, dtype=jnp.int32) * 37) % N
    rowp = jnp.concatenate([row, pidx]).reshape(EP // CH, CH)
    colp = jnp.concatenate([col, pidx]).reshape(EP // CH, CH)
    ewp = jnp.concatenate([ew, jnp.zeros((pad,), jnp.float32)]
                          ).reshape(EP // CH, CH)
    xp = jnp.concatenate(
        [x, jnp.zeros((NP - N, x.shape[1]), jnp.float32)], axis=0)
    F2P = 64
    W2p = jnp.concatenate(
        [W2, jnp.zeros((W2.shape[0], F2P - W2.shape[1]), jnp.float32)], axis=1)

    sp, deg = _prep_call(rowp, colp, ewp)
    degc = deg.reshape(NP, 1)
    g1 = _mm_call(xp, W1)
    agg1 = _agg_call([g1[:, :64], g1[:, 64:]], rowp, colp, sp)
    g2 = _mid_call(degc, agg1[0, 0], agg1[0, 1], agg1[1, 0], agg1[1, 1],
                   g1, b1.reshape(1, -1), W2p)
    agg2 = _agg_call([g2], rowp, colp, sp)
    out = _final_call(degc, agg2[0, 0], agg2[0, 1], g2, b2.reshape(1, -1))
    return out[:N]


# speedup vs baseline: 2.2099x; 1.0662x over previous
import functools

import jax
import jax.numpy as jnp
from jax import lax
from jax.experimental import pallas as pl
from jax.experimental.pallas import tpu as pltpu
from jax.experimental.pallas import tpu_sc as plsc

N = 10000
NP = 10240
EP = 327680
CH = 128
EPW = EP // 32
NCH = EPW // CH
NC, NS, L = 2, 16, 16
RPT = NP // NS


def _mesh():
    return plsc.VectorSubcoreMesh(
        core_axis_name="c", subcore_axis_name="s",
        num_cores=NC, num_subcores=NS)


_SC_PARAMS = pltpu.CompilerParams(
    needs_layout_passes=False, use_tc_tiling_on_sc=False)


def _rsqrt16(x):
    i = lax.bitcast_convert_type(x, jnp.int32)
    i = jnp.full((L,), 0x5F3759DF, jnp.int32) - lax.shift_right_logical(i, 1)
    y = lax.bitcast_convert_type(i, jnp.float32)
    for _ in range(3):
        y = y * (1.5 - 0.5 * x * y * y)
    return y


def _prep_call(rowf, colf, ewf):
    EPT = EP // NS

    @functools.partial(
        pl.kernel,
        out_type=(jax.ShapeDtypeStruct((EP,), jnp.float32),
                  jax.ShapeDtypeStruct((NP,), jnp.float32)),
        mesh=_mesh(),
        compiler_params=_SC_PARAMS,
        scratch_types=[
            pltpu.VMEM((EPT,), jnp.int32),
            pltpu.VMEM((EPT,), jnp.float32),
            pltpu.VMEM((EPW,), jnp.int32),
            pltpu.VMEM((EPW,), jnp.int32),
            pltpu.VMEM((EPW,), jnp.float32),
            pltpu.VMEM((NP,), jnp.float32),
            pltpu.VMEM((NP,), jnp.float32),
            pltpu.VMEM((RPT,), jnp.float32),
            pltpu.VMEM_SHARED((NP,), jnp.float32),
            pltpu.SemaphoreType.DMA,
        ],
    )
    def prep_kernel(row_hbm, col_hbm, ew_hbm, s_hbm, deg_hbm,
                    dcol_v, dew_v, row_v, col_v, ew_v, deg_v, dinv_v, zb_v,
                    acc_sh, sem):
        cid = lax.axis_index("c")
        sid = lax.axis_index("s")
        wid = cid * NS + sid

        pltpu.sync_copy(col_hbm.at[pl.ds(sid * EPT, EPT)], dcol_v)
        pltpu.sync_copy(ew_hbm.at[pl.ds(sid * EPT, EPT)], dew_v)

        def zb(k, carry):
            zb_v[pl.ds(k * L, L)] = jnp.zeros((L,), jnp.float32)
            return carry
        lax.fori_loop(0, RPT // L, zb, 0)
        pltpu.sync_copy(zb_v, acc_sh.at[pl.ds(sid * RPT, RPT)])
        plsc.subcore_barrier()

        K = 8

        def dchunk(k, carry):
            for j in range(K):
                o = (k * K + j) * CH
                pltpu.async_copy(dew_v.at[pl.ds(o, CH)],
                                 acc_sh.at[dcol_v.at[pl.ds(o, CH)]], sem,
                                 add=True)
            for j in range(K):
                o = (k * K + j) * CH
                pltpu.make_async_copy(
                    dew_v.at[pl.ds(o, CH)],
                    acc_sh.at[dcol_v.at[pl.ds(o, CH)]], sem).wait()
            return carry
        lax.fori_loop(0, EPT // CH // K, dchunk, 0)
        plsc.subcore_barrier()

        @pl.when(cid == 0)
        def _():
            pltpu.sync_copy(acc_sh.at[pl.ds(sid * RPT, RPT)],
                            deg_hbm.at[pl.ds(sid * RPT, RPT)])

        pltpu.sync_copy(acc_sh, deg_v)
        pltpu.sync_copy(row_hbm.at[pl.ds(wid * EPW, EPW)], row_v)
        pltpu.sync_copy(col_hbm.at[pl.ds(wid * EPW, EPW)], col_v)
        pltpu.sync_copy(ew_hbm.at[pl.ds(wid * EPW, EPW)], ew_v)

        def dbody(k, carry):
            sl = pl.ds(k * L, L)
            d = deg_v[sl] + 1.0
            dinv_v[sl] = _rsqrt16(d)
            return carry
        lax.fori_loop(0, NP // L, dbody, 0)

        def nchunk(k, carry):
            for sub in range(4):
                sl = pl.ds(k * 4 * L + sub * L, L)
                rr = row_v[sl]
                cc = col_v[sl]
                w = ew_v[sl]
                ew_v[sl] = (plsc.load_gather(dinv_v, [rr]) * w *
                            plsc.load_gather(dinv_v, [cc]))
            return carry
        lax.fori_loop(0, EPW // (4 * L), nchunk, 0)
        pltpu.sync_copy(ew_v, s_hbm.at[pl.ds(wid * EPW, EPW)])

    return prep_kernel(rowf, colf, ewf)


def _agg_call(F, tables, rowf, colf, sf):
    NT = len(tables)

    @functools.partial(
        pl.kernel,
        out_type=jax.ShapeDtypeStruct((NT, NC, NP, F), jnp.float32),
        mesh=_mesh(),
        compiler_params=_SC_PARAMS,
        scratch_types=[
            pltpu.VMEM((EPW,), jnp.int32),
            pltpu.VMEM((EPW,), jnp.int32),
            pltpu.VMEM((EPW,), jnp.float32),
            pltpu.VMEM((2, CH, F), jnp.float32),
            pltpu.VMEM((2, CH, F), jnp.float32),
            pltpu.VMEM_SHARED((NP, F), jnp.float32),
            pltpu.SemaphoreType.DMA,
            pltpu.SemaphoreType.DMA,
            pltpu.SemaphoreType.DMA,
            pltpu.SemaphoreType.DMA,
        ],
    )
    def agg_kernel(*refs):
        g_hbms = refs[:NT]
        row_hbm, col_hbm, s_hbm, out_hbm = refs[NT:NT + 4]
        (row_v, col_v, s_v, gbuf, sbuf, acc_sh,
         sg0, sg1, ss0, ss1) = refs[NT + 4:]
        cid = lax.axis_index("c")
        sid = lax.axis_index("s")
        wid = cid * NS + sid
        semg = (sg0, sg1)
        sems = (ss0, ss1)

        pltpu.sync_copy(row_hbm.at[pl.ds(wid * EPW, EPW)], row_v)
        pltpu.sync_copy(col_hbm.at[pl.ds(wid * EPW, EPW)], col_v)
        pltpu.sync_copy(s_hbm.at[pl.ds(wid * EPW, EPW)], s_v)

        def issue_gather(g_hbm, b, ch):
            pltpu.async_copy(g_hbm.at[row_v.at[pl.ds(ch * CH, CH)]],
                             gbuf.at[b], semg[b])

        def wait_gather(g_hbm, b, ch):
            pltpu.make_async_copy(
                g_hbm.at[row_v.at[pl.ds(ch * CH, CH)]],
                gbuf.at[b], semg[b]).wait()

        def issue_scatter(b, ch):
            pltpu.async_copy(sbuf.at[b],
                             acc_sh.at[col_v.at[pl.ds(ch * CH, CH)]],
                             sems[b], add=True)

        def wait_scatter(b, ch):
            pltpu.make_async_copy(
                sbuf.at[b], acc_sh.at[col_v.at[pl.ds(ch * CH, CH)]],
                sems[b]).wait()

        def scale(b, ch):
            R = 4

            def rbody(r, carry):
                rows = [r * R + rr for rr in range(R)]
                sbs = [plsc.load_gather(
                    s_v, [jnp.full((L,), ch * CH + row, jnp.int32)])
                       for row in rows]
                vals = [[gbuf[b, row, pl.ds(gg * L, L)]
                         for gg in range(F // L)] for row in rows]
                for rr, row in enumerate(rows):
                    for gg in range(F // L):
                        sbuf[b, row, pl.ds(gg * L, L)] = vals[rr][gg] * sbs[rr]
                return carry
            lax.fori_loop(0, CH // R, rbody, 0)

        for t, g_hbm in enumerate(g_hbms):
            def zrow(r, carry):
                for gg in range(F // L):
                    gbuf[0, r, pl.ds(gg * L, L)] = jnp.zeros((L,),
                                                            jnp.float32)
                return carry
            lax.fori_loop(0, CH, zrow, 0)
            for k in range(RPT // CH):
                pltpu.sync_copy(gbuf.at[0],
                                acc_sh.at[pl.ds(sid * RPT + k * CH, CH)])
            plsc.subcore_barrier()

            for b in range(2):
                issue_gather(g_hbm, b, b)
            for b in range(2):
                wait_gather(g_hbm, b, b)
                scale(b, b)
                issue_scatter(b, b)
                issue_gather(g_hbm, b, b + 2)

            def step(k, carry):
                for b in range(2):
                    ch = 2 * k + b
                    wait_gather(g_hbm, b, ch)
                    wait_scatter(b, ch - 2)
                    scale(b, ch)
                    issue_scatter(b, ch)
                    issue_gather(g_hbm, b, ch + 2)
                return carry
            lax.fori_loop(1, NCH // 2 - 1, step, 0)

            for b in range(2):
                ch = NCH - 2 + b
                wait_gather(g_hbm, b, ch)
                wait_scatter(b, ch - 2)
                scale(b, ch)
                issue_scatter(b, ch)
            for b in range(2):
                wait_scatter(b, NCH - 2 + b)

            plsc.subcore_barrier()
            pltpu.sync_copy(acc_sh.at[pl.ds(sid * RPT, RPT)],
                            out_hbm.at[t, cid, pl.ds(sid * RPT, RPT)])
            if t + 1 < NT:
                plsc.subcore_barrier()

    return agg_kernel(*tables, rowf, colf, sf)


def _mm_call(x, w):
    def body(x_ref, w_ref, oa_ref, ob_ref):
        o = jnp.dot(x_ref[...], w_ref[...],
                    preferred_element_type=jnp.float32)
        oa_ref[...] = o[:, :64]
        ob_ref[...] = o[:, 64:]
    return pl.pallas_call(
        body,
        out_shape=(jax.ShapeDtypeStruct((x.shape[0], 64), jnp.float32),
                   jax.ShapeDtypeStruct((x.shape[0], 64), jnp.float32)),
    )(x, w)


def _mid_call(degc, agg1, g1a, g1b, b1r, W2p):
    def body(d_ref, a_ref, ga_ref, gb_ref, b_ref, w_ref, o_ref):
        inv = 1.0 / (d_ref[...] + 1.0)
        agg = jnp.concatenate(
            [a_ref[0, 0] + a_ref[0, 1], a_ref[1, 0] + a_ref[1, 1]], axis=1)
        g = jnp.concatenate([ga_ref[...], gb_ref[...]], axis=1)
        z = agg + g * inv + b_ref[...]
        h = jnp.maximum(z, 0.0)
        o_ref[...] = jnp.dot(h, w_ref[...],
                             preferred_element_type=jnp.float32)
    return pl.pallas_call(
        body,
        out_shape=jax.ShapeDtypeStruct((NP, W2p.shape[1]), jnp.float32),
    )(degc, agg1, g1a, g1b, b1r, W2p)


def _final_call(degc, agg2, g2, b2r):
    F2 = b2r.shape[1]
    def body(d_ref, a_ref, g_ref, b_ref, o_ref):
        inv = 1.0 / (d_ref[...] + 1.0)
        z = (a_ref[0, 0] + a_ref[0, 1] + g_ref[...] * inv)[:, :F2] + b_ref[...]
        m = jnp.max(z, axis=1, keepdims=True)
        e = jnp.exp(z - m)
        s = jnp.sum(e, axis=1, keepdims=True)
        o_ref[...] = z - m - jnp.log(s)
    return pl.pallas_call(
        body,
        out_shape=jax.ShapeDtypeStruct((NP, F2), jnp.float32),
    )(degc, agg2, g2, b2r)


def kernel(x, edge_index, edge_weight, W1, b1, W2, b2):
    row = edge_index[0].astype(jnp.int32)
    col = edge_index[1].astype(jnp.int32)
    ew = edge_weight.astype(jnp.float32)
    pad = EP - row.shape[0]
    pidx = (jnp.arange(pad, dtype=jnp.int32) * 37) % N
    rowf = jnp.concatenate([row, pidx])
    colf = jnp.concatenate([col, pidx])
    ewf = jnp.concatenate([ew, jnp.zeros((pad,), jnp.float32)])
    xp = jnp.concatenate(
        [x, jnp.zeros((NP - N, x.shape[1]), jnp.float32)], axis=0)
    F2P = 48
    W2p = jnp.concatenate(
        [W2, jnp.zeros((W2.shape[0], F2P - W2.shape[1]), jnp.float32)], axis=1)

    sf, deg = _prep_call(rowf, colf, ewf)
    degc = deg.reshape(NP, 1)
    g1a, g1b = _mm_call(xp, W1)
    agg1 = _agg_call(64, [g1a, g1b], rowf, colf, sf)
    g2 = _mid_call(degc, agg1, g1a, g1b,
                   b1.reshape(1, -1), W2p)
    agg2 = _agg_call(F2P, [g2], rowf, colf, sf)
    out = _final_call(degc, agg2, g2, b2.reshape(1, -1))
    return out[:N]
